# baseline (speedup 1.0000x reference)
import functools

import jax
import jax.numpy as jnp
from jax import lax
from jax.experimental import pallas as pl
from jax.experimental.pallas import tpu as pltpu
from jax.experimental.pallas import tpu_sc as plsc

_BR = 8000

_NC = 2
_NS = 16
_NW = _NC * _NS


def _tc_copy_kernel(t_in, t_out):
    t_out[...] = t_in[...]


def _tc_copy(x):
    m, n = x.shape
    spec = pl.BlockSpec((_BR, n), lambda i: (i, 0))
    return pl.pallas_call(
        _tc_copy_kernel,
        grid=(m // _BR,),
        in_specs=[spec],
        out_specs=spec,
        out_shape=jax.ShapeDtypeStruct((m, n), x.dtype),
    )(x)


def _sc_copy(x):
    m, n = x.shape
    chunk = ((m + _NW - 1) // _NW + 7) // 8 * 8
    last = m - (_NW - 1) * chunk
    mesh = plsc.VectorSubcoreMesh(core_axis_name="c", subcore_axis_name="s")

    @functools.partial(
        pl.kernel,
        out_type=jax.ShapeDtypeStruct((m, n), x.dtype),
        mesh=mesh,
    )
    def body(in_hbm, out_hbm):
        wid = lax.axis_index("s") * _NC + lax.axis_index("c")
        base = wid * chunk

        @pl.when(wid < _NW - 1)
        def _():
            sl = pl.ds(base, chunk)
            pltpu.sync_copy(in_hbm.at[sl], out_hbm.at[sl])

        @pl.when(wid == _NW - 1)
        def _():
            sl = pl.ds(base, last)
            pltpu.sync_copy(in_hbm.at[sl], out_hbm.at[sl])

    return body(x)


def kernel(bbox_targets, bbox_weights, labels):
    del labels
    return (_tc_copy(bbox_targets), _sc_copy(bbox_weights))


# speedup vs baseline: 10.0751x; 10.0751x over previous
import functools

import jax
import jax.numpy as jnp
from jax import lax
from jax.experimental import pallas as pl
from jax.experimental.pallas import tpu as pltpu
from jax.experimental.pallas import tpu_sc as plsc

_BR = 8000

_NC = 2
_NS = 16
_NW = _NC * _NS
_SUB = 1000


def _tc_copy_kernel(t_in, t_out):
    t_out[...] = t_in[...]


def _tc_copy(x):
    m, n = x.shape
    spec = pl.BlockSpec((_BR, n), lambda i: (i, 0))
    return pl.pallas_call(
        _tc_copy_kernel,
        grid=(m // _BR,),
        in_specs=[spec],
        out_specs=spec,
        out_shape=jax.ShapeDtypeStruct((m, n), x.dtype),
    )(x)


def _sc_copy(x):
    m, n = x.shape
    chunk = ((m + _NW - 1) // _NW + 7) // 8 * 8
    last = m - (_NW - 1) * chunk
    n_full, tail = divmod(chunk, _SUB)
    n_full_last, tail_last = divmod(last, _SUB)
    mesh = plsc.VectorSubcoreMesh(core_axis_name="c", subcore_axis_name="s")

    @functools.partial(
        pl.kernel,
        out_type=jax.ShapeDtypeStruct((m, n), x.dtype),
        mesh=mesh,
        scratch_types=[pltpu.VMEM((_SUB, n), x.dtype)],
    )
    def body(in_hbm, out_hbm, buf):
        wid = lax.axis_index("s") * _NC + lax.axis_index("c")
        base = wid * chunk

        def move(start, size):
            sl = pl.ds(start, size)
            pltpu.sync_copy(in_hbm.at[sl], buf.at[pl.ds(0, size)])
            pltpu.sync_copy(buf.at[pl.ds(0, size)], out_hbm.at[sl])

        @pl.when(wid < _NW - 1)
        def _():
            def step(j, _):
                move(base + j * _SUB, _SUB)
                return ()
            lax.fori_loop(0, n_full, step, ())
            if tail:
                move(base + n_full * _SUB, tail)

        @pl.when(wid == _NW - 1)
        def _():
            def step(j, _):
                move(base + j * _SUB, _SUB)
                return ()
            lax.fori_loop(0, n_full_last, step, ())
            if tail_last:
                move(base + n_full_last * _SUB, tail_last)

    return body(x)


def kernel(bbox_targets, bbox_weights, labels):
    del labels
    return (_tc_copy(bbox_targets), _sc_copy(bbox_weights))
